# baseline (speedup 1.0000x reference)
import functools

import jax
import jax.numpy as jnp
from jax import lax
from jax.experimental import pallas as pl
from jax.experimental.pallas import tpu as pltpu
from jax.experimental.pallas import tpu_sc as plsc

N = 10000
E = 320000
NC = 2
NS = 16
NW = NC * NS
EBA = 80
NSTEP0 = 222
NSTEP1 = 28
EBC = 128
NSTEPC = 80
EPADC = NW * NSTEPC * EBC
NP = 10240
RPW = NP // NS

@functools.lru_cache(maxsize=None)
def _mesh():
    return plsc.VectorSubcoreMesh(core_axis_name="c", subcore_axis_name="s",
                                  num_cores=NC, num_subcores=NS)


def _sc_agg_body(D, m_hbm, src_hbm, dst_hbm, zeros_hbm, out_hbm,
                 isA, isB, idA, idB, rowsA, rowsB, acc,
                 sgA, sgB, siA, siB):
    c = lax.axis_index("c")
    s = lax.axis_index("s")
    nst = jnp.where(c == 0, NSTEP0, NSTEP1)
    base = c * NS * NSTEP0 * EBA + s * nst * EBA

    pltpu.sync_copy(zeros_hbm, acc.at[pl.ds(s * RPW, RPW)])
    pltpu.sync_copy(src_hbm.at[pl.ds(base, EBA)], isA)
    pltpu.sync_copy(dst_hbm.at[pl.ds(base, EBA)], idA)
    pltpu.sync_copy(src_hbm.at[pl.ds(base + EBA, EBA)], isB)
    pltpu.sync_copy(dst_hbm.at[pl.ds(base + EBA, EBA)], idB)
    plsc.subcore_barrier()

    pltpu.async_copy(m_hbm.at[isA], rowsA, sgA)
    pltpu.async_copy(m_hbm.at[isB], rowsB, sgB)

    def pair(k, carry):
        b = 2 * k
        offA = base + (b + 2) * EBA
        offB = base + (b + 3) * EBA
        pltpu.make_async_copy(m_hbm.at[isA], rowsA, sgA).wait()
        pltpu.sync_copy(rowsA, acc.at[idA], add=True)
        pltpu.async_copy(src_hbm.at[pl.ds(offA, EBA)], isA, siA)
        pltpu.async_copy(dst_hbm.at[pl.ds(offA, EBA)], idA, siA)
        pltpu.make_async_copy(m_hbm.at[isB], rowsB, sgB).wait()
        pltpu.sync_copy(rowsB, acc.at[idB], add=True)
        pltpu.async_copy(src_hbm.at[pl.ds(offB, EBA)], isB, siB)
        pltpu.async_copy(dst_hbm.at[pl.ds(offB, EBA)], idB, siB)
        pltpu.make_async_copy(src_hbm.at[pl.ds(offA, EBA)], isA, siA).wait()
        pltpu.make_async_copy(dst_hbm.at[pl.ds(offA, EBA)], idA, siA).wait()
        pltpu.async_copy(m_hbm.at[isA], rowsA, sgA)
        pltpu.make_async_copy(src_hbm.at[pl.ds(offB, EBA)], isB, siB).wait()
        pltpu.make_async_copy(dst_hbm.at[pl.ds(offB, EBA)], idB, siB).wait()
        pltpu.async_copy(m_hbm.at[isB], rowsB, sgB)
        return carry

    lax.fori_loop(0, nst // 2 - 1, pair, 0)
    pltpu.make_async_copy(m_hbm.at[isA], rowsA, sgA).wait()
    pltpu.sync_copy(rowsA, acc.at[idA], add=True)
    pltpu.make_async_copy(m_hbm.at[isB], rowsB, sgB).wait()
    pltpu.sync_copy(rowsB, acc.at[idB], add=True)
    plsc.subcore_barrier()
    pltpu.sync_copy(acc.at[pl.ds(s * RPW, RPW)],
                    out_hbm.at[c, pl.ds(s * RPW, RPW)])


@functools.lru_cache(maxsize=None)
def _make_sc_agg(D):
    return pl.kernel(
        functools.partial(_sc_agg_body, D),
        out_type=jax.ShapeDtypeStruct((NC, NP, D), jnp.float32),
        mesh=_mesh(),
        scratch_types=[
            pltpu.VMEM((EBA,), jnp.int32),
            pltpu.VMEM((EBA,), jnp.int32),
            pltpu.VMEM((EBA,), jnp.int32),
            pltpu.VMEM((EBA,), jnp.int32),
            pltpu.VMEM((EBA, D), jnp.float32),
            pltpu.VMEM((EBA, D), jnp.float32),
            pltpu.VMEM_SHARED((NP, D), jnp.float32),
            pltpu.SemaphoreType.DMA,
            pltpu.SemaphoreType.DMA,
            pltpu.SemaphoreType.DMA,
            pltpu.SemaphoreType.DMA,
        ],
    )


def _sc_count_body(dst3_hbm, ones_hbm, zeros_hbm, out_hbm,
                   idx_d, ones_v, acc, ssA, ssB):
    c = lax.axis_index("c")
    s = lax.axis_index("s")
    wid = c * NS + s
    pltpu.sync_copy(zeros_hbm, acc.at[pl.ds(s * RPW, RPW)])
    pltpu.sync_copy(ones_hbm, ones_v)
    pltpu.sync_copy(dst3_hbm.at[wid], idx_d)
    plsc.subcore_barrier()

    pltpu.async_copy(ones_v, acc.at[idx_d.at[0]], ssA, add=True)
    pltpu.async_copy(ones_v, acc.at[idx_d.at[1]], ssB, add=True)

    def pair(k, carry):
        b = 2 * k
        pltpu.make_async_copy(ones_v, acc.at[idx_d.at[b]], ssA).wait()
        pltpu.async_copy(ones_v, acc.at[idx_d.at[b + 2]], ssA, add=True)
        pltpu.make_async_copy(ones_v, acc.at[idx_d.at[b]], ssB).wait()
        pltpu.async_copy(ones_v, acc.at[idx_d.at[b + 3]], ssB, add=True)
        return carry

    lax.fori_loop(0, NSTEPC // 2 - 1, pair, 0)
    pltpu.make_async_copy(ones_v, acc.at[idx_d.at[0]], ssA).wait()
    pltpu.make_async_copy(ones_v, acc.at[idx_d.at[0]], ssB).wait()
    plsc.subcore_barrier()
    pltpu.sync_copy(acc.at[pl.ds(s * RPW, RPW)],
                    out_hbm.at[c, pl.ds(s * RPW, RPW)])


@functools.lru_cache(maxsize=None)
def _make_sc_count():
    return pl.kernel(
        _sc_count_body,
        out_type=jax.ShapeDtypeStruct((NC, NP, 128), jnp.float32),
        mesh=_mesh(),
        scratch_types=[
            pltpu.VMEM((NSTEPC, EBC), jnp.int32),
            pltpu.VMEM((EBC, 128), jnp.float32),
            pltpu.VMEM_SHARED((NP, 128), jnp.float32),
            pltpu.SemaphoreType.DMA,
            pltpu.SemaphoreType.DMA,
        ],
    )


RB = 1000
GRID = N // RB


def _proj_body(x_ref, win_ref, bin_ref, wl_ref, bl_ref, wr_ref,
               inp_ref, m_ref, r_ref):
    inp = jnp.dot(x_ref[...], win_ref[...],
                  preferred_element_type=jnp.float32) + bin_ref[...]
    h = jnp.maximum(inp, 0.0)
    inp_ref[...] = inp
    m_ref[...] = jnp.dot(h, wl_ref[...], preferred_element_type=jnp.float32)
    r_ref[...] = jnp.dot(h, wr_ref[...],
                         preferred_element_type=jnp.float32) + bl_ref[...]


def _tc_proj(x, W_in, b_in, Wl0, bl0, Wr0):
    row = pl.BlockSpec((RB, 128), lambda i: (i, 0))
    full = pl.BlockSpec((128, 128), lambda i: (0, 0))
    bias = pl.BlockSpec((1, 128), lambda i: (0, 0))
    return pl.pallas_call(
        _proj_body,
        grid=(GRID,),
        in_specs=[row, full, bias, full, bias, full],
        out_specs=[row, row, row],
        out_shape=[jax.ShapeDtypeStruct((N, 128), jnp.float32)] * 3,
    )(x, W_in, b_in.reshape(1, 128), Wl0, bl0.reshape(1, 128), Wr0)


def _inv_counts(cnt_ref):
    cnt = cnt_ref[0, :, 0:1] + cnt_ref[1, :, 0:1]
    return 1.0 / jnp.maximum(cnt, 1.0)


def _combine_body(do, s_ref, cnt_ref, r_ref, inp_ref, wl_ref, bl_ref, wr_ref,
                  m_ref, rn_ref):
    agg = (s_ref[0] + s_ref[1]) * _inv_counts(cnt_ref)
    h = jnp.maximum(agg + r_ref[...], 0.0) + 0.2 * inp_ref[...]
    m_ref[...] = jnp.dot(h, wl_ref[...], preferred_element_type=jnp.float32)
    rn_ref[...] = jnp.dot(h, wr_ref[...],
                          preferred_element_type=jnp.float32) + bl_ref[...]


def _tc_combine(S, cntP, r, inp, Wl, bl, Wr):
    do = Wl.shape[1]
    row = pl.BlockSpec((RB, 128), lambda i: (i, 0))
    rowo = pl.BlockSpec((RB, do), lambda i: (i, 0))
    return pl.pallas_call(
        functools.partial(_combine_body, do),
        grid=(GRID,),
        in_specs=[
            pl.BlockSpec((2, RB, 128), lambda i: (0, i, 0)),
            pl.BlockSpec((2, RB, 128), lambda i: (0, i, 0)),
            row, row,
            pl.BlockSpec((128, do), lambda i: (0, 0)),
            pl.BlockSpec((1, do), lambda i: (0, 0)),
            pl.BlockSpec((128, do), lambda i: (0, 0)),
        ],
        out_specs=[rowo, rowo],
        out_shape=[jax.ShapeDtypeStruct((N, do), jnp.float32)] * 2,
    )(S, cntP, r, inp, Wl, bl.reshape(1, do), Wr)


def _final_body(s_ref, cnt_ref, r_ref, out_ref):
    zf = (s_ref[0] + s_ref[1]) * _inv_counts(cnt_ref) + r_ref[...]
    z = zf[:, 0:64]
    zmax = jnp.max(z, axis=-1, keepdims=True)
    ez = jnp.exp(z - zmax)
    lse = jnp.log(jnp.sum(ez, axis=-1, keepdims=True)) + zmax
    out_ref[...] = z - lse


def _tc_final(S, cntP, r):
    row = pl.BlockSpec((RB, 128), lambda i: (i, 0))
    return pl.pallas_call(
        _final_body,
        grid=(GRID,),
        in_specs=[
            pl.BlockSpec((2, RB, 128), lambda i: (0, i, 0)),
            pl.BlockSpec((2, RB, 128), lambda i: (0, i, 0)),
            row,
        ],
        out_specs=pl.BlockSpec((RB, 64), lambda i: (i, 0)),
        out_shape=jax.ShapeDtypeStruct((N, 64), jnp.float32),
    )(S, cntP, r)


def kernel(x, edge_index, W_in, b_in, Wl, bl, Wr):
    src = edge_index[0]
    dst = edge_index[1]
    dst3 = jnp.concatenate(
        [dst, jnp.full((EPADC - E,), NP - 1, jnp.int32)]
    ).reshape(NW, NSTEPC, EBC)
    zeros128 = jnp.zeros((RPW, 128), jnp.float32)
    ones128 = jnp.ones((EBC, 128), jnp.float32)
    Wls = [Wl[1], Wl[2], jnp.pad(Wl[3], ((0, 0), (0, 64)))]
    Wrs = [Wr[1], Wr[2], jnp.pad(Wr[3], ((0, 0), (0, 64)))]
    bls = [bl[1], bl[2], jnp.pad(bl[3], (0, 64))]

    cntP = _make_sc_count()(dst3, ones128, zeros128)
    inp, m, r = _tc_proj(x, W_in, b_in, Wl[0], bl[0], Wr[0])
    for i in range(3):
        S = _make_sc_agg(128)(m, src, dst, zeros128)
        m, r = _tc_combine(S, cntP, r, inp, Wls[i], bls[i], Wrs[i])
    S = _make_sc_agg(128)(m, src, dst, zeros128)
    return _tc_final(S, cntP, r)


# speedup vs baseline: 1.0815x; 1.0815x over previous
import functools

import jax
import jax.numpy as jnp
from jax import lax
from jax.experimental import pallas as pl
from jax.experimental.pallas import tpu as pltpu
from jax.experimental.pallas import tpu_sc as plsc

N = 10000
E = 320000
NC = 2
NS = 16
NW = NC * NS
EBA = 80
NSTEP0 = 200
NSTEP1 = 50
EBC = 128
NSTEPC = 80
EPADC = NW * NSTEPC * EBC
NP = 10240
RPW = NP // NS

@functools.lru_cache(maxsize=None)
def _mesh():
    return plsc.VectorSubcoreMesh(core_axis_name="c", subcore_axis_name="s",
                                  num_cores=NC, num_subcores=NS)


def _sc_agg_body(D, m_hbm, src_hbm, dst_hbm, zeros_hbm, out_hbm,
                 isA, isB, idA, idB, rowsA, rowsB, acc,
                 sgA, sgB, siA, siB):
    c = lax.axis_index("c")
    s = lax.axis_index("s")
    nst = jnp.where(c == 0, NSTEP0, NSTEP1)
    base = c * NS * NSTEP0 * EBA + s * nst * EBA

    pltpu.sync_copy(zeros_hbm, acc.at[pl.ds(s * RPW, RPW)])
    pltpu.sync_copy(src_hbm.at[pl.ds(base, EBA)], isA)
    pltpu.sync_copy(dst_hbm.at[pl.ds(base, EBA)], idA)
    pltpu.sync_copy(src_hbm.at[pl.ds(base + EBA, EBA)], isB)
    pltpu.sync_copy(dst_hbm.at[pl.ds(base + EBA, EBA)], idB)
    plsc.subcore_barrier()

    pltpu.async_copy(m_hbm.at[isA], rowsA, sgA)
    pltpu.async_copy(m_hbm.at[isB], rowsB, sgB)

    def pair(k, carry):
        b = 2 * k
        offA = base + (b + 2) * EBA
        offB = base + (b + 3) * EBA
        pltpu.make_async_copy(m_hbm.at[isA], rowsA, sgA).wait()
        pltpu.sync_copy(rowsA, acc.at[idA], add=True)
        pltpu.async_copy(src_hbm.at[pl.ds(offA, EBA)], isA, siA)
        pltpu.async_copy(dst_hbm.at[pl.ds(offA, EBA)], idA, siA)
        pltpu.make_async_copy(m_hbm.at[isB], rowsB, sgB).wait()
        pltpu.sync_copy(rowsB, acc.at[idB], add=True)
        pltpu.async_copy(src_hbm.at[pl.ds(offB, EBA)], isB, siB)
        pltpu.async_copy(dst_hbm.at[pl.ds(offB, EBA)], idB, siB)
        pltpu.make_async_copy(src_hbm.at[pl.ds(offA, EBA)], isA, siA).wait()
        pltpu.make_async_copy(dst_hbm.at[pl.ds(offA, EBA)], idA, siA).wait()
        pltpu.async_copy(m_hbm.at[isA], rowsA, sgA)
        pltpu.make_async_copy(src_hbm.at[pl.ds(offB, EBA)], isB, siB).wait()
        pltpu.make_async_copy(dst_hbm.at[pl.ds(offB, EBA)], idB, siB).wait()
        pltpu.async_copy(m_hbm.at[isB], rowsB, sgB)
        return carry

    lax.fori_loop(0, nst // 2 - 1, pair, 0)
    pltpu.make_async_copy(m_hbm.at[isA], rowsA, sgA).wait()
    pltpu.sync_copy(rowsA, acc.at[idA], add=True)
    pltpu.make_async_copy(m_hbm.at[isB], rowsB, sgB).wait()
    pltpu.sync_copy(rowsB, acc.at[idB], add=True)
    plsc.subcore_barrier()
    pltpu.sync_copy(acc.at[pl.ds(s * RPW, RPW)],
                    out_hbm.at[c, pl.ds(s * RPW, RPW)])


@functools.lru_cache(maxsize=None)
def _make_sc_agg(D):
    return pl.kernel(
        functools.partial(_sc_agg_body, D),
        out_type=jax.ShapeDtypeStruct((NC, NP, D), jnp.float32),
        mesh=_mesh(),
        scratch_types=[
            pltpu.VMEM((EBA,), jnp.int32),
            pltpu.VMEM((EBA,), jnp.int32),
            pltpu.VMEM((EBA,), jnp.int32),
            pltpu.VMEM((EBA,), jnp.int32),
            pltpu.VMEM((EBA, D), jnp.float32),
            pltpu.VMEM((EBA, D), jnp.float32),
            pltpu.VMEM_SHARED((NP, D), jnp.float32),
            pltpu.SemaphoreType.DMA,
            pltpu.SemaphoreType.DMA,
            pltpu.SemaphoreType.DMA,
            pltpu.SemaphoreType.DMA,
        ],
    )


def _sc_count_body(dst3_hbm, ones_hbm, zeros_hbm, out_hbm,
                   idx_d, ones_v, acc, ssA, ssB):
    c = lax.axis_index("c")
    s = lax.axis_index("s")
    wid = c * NS + s
    pltpu.sync_copy(zeros_hbm, acc.at[pl.ds(s * RPW, RPW)])
    pltpu.sync_copy(ones_hbm, ones_v)
    pltpu.sync_copy(dst3_hbm.at[wid], idx_d)
    plsc.subcore_barrier()

    pltpu.async_copy(ones_v, acc.at[idx_d.at[0]], ssA, add=True)
    pltpu.async_copy(ones_v, acc.at[idx_d.at[1]], ssB, add=True)

    def pair(k, carry):
        b = 2 * k
        pltpu.make_async_copy(ones_v, acc.at[idx_d.at[b]], ssA).wait()
        pltpu.async_copy(ones_v, acc.at[idx_d.at[b + 2]], ssA, add=True)
        pltpu.make_async_copy(ones_v, acc.at[idx_d.at[b]], ssB).wait()
        pltpu.async_copy(ones_v, acc.at[idx_d.at[b + 3]], ssB, add=True)
        return carry

    lax.fori_loop(0, NSTEPC // 2 - 1, pair, 0)
    pltpu.make_async_copy(ones_v, acc.at[idx_d.at[0]], ssA).wait()
    pltpu.make_async_copy(ones_v, acc.at[idx_d.at[0]], ssB).wait()
    plsc.subcore_barrier()
    pltpu.sync_copy(acc.at[pl.ds(s * RPW, RPW)],
                    out_hbm.at[c, pl.ds(s * RPW, RPW)])


@functools.lru_cache(maxsize=None)
def _make_sc_count():
    return pl.kernel(
        _sc_count_body,
        out_type=jax.ShapeDtypeStruct((NC, NP, 128), jnp.float32),
        mesh=_mesh(),
        scratch_types=[
            pltpu.VMEM((NSTEPC, EBC), jnp.int32),
            pltpu.VMEM((EBC, 128), jnp.float32),
            pltpu.VMEM_SHARED((NP, 128), jnp.float32),
            pltpu.SemaphoreType.DMA,
            pltpu.SemaphoreType.DMA,
        ],
    )


RB = 1000
GRID = N // RB


def _proj_body(x_ref, win_ref, bin_ref, wl_ref, bl_ref, wr_ref,
               inp_ref, m_ref, r_ref):
    inp = jnp.dot(x_ref[...], win_ref[...],
                  preferred_element_type=jnp.float32) + bin_ref[...]
    h = jnp.maximum(inp, 0.0)
    inp_ref[...] = inp
    m_ref[...] = jnp.dot(h, wl_ref[...], preferred_element_type=jnp.float32)
    r_ref[...] = jnp.dot(h, wr_ref[...],
                         preferred_element_type=jnp.float32) + bl_ref[...]


def _tc_proj(x, W_in, b_in, Wl0, bl0, Wr0):
    row = pl.BlockSpec((RB, 128), lambda i: (i, 0))
    full = pl.BlockSpec((128, 128), lambda i: (0, 0))
    bias = pl.BlockSpec((1, 128), lambda i: (0, 0))
    return pl.pallas_call(
        _proj_body,
        grid=(GRID,),
        in_specs=[row, full, bias, full, bias, full],
        out_specs=[row, row, row],
        out_shape=[jax.ShapeDtypeStruct((N, 128), jnp.float32)] * 3,
    )(x, W_in, b_in.reshape(1, 128), Wl0, bl0.reshape(1, 128), Wr0)


def _inv_counts(cnt_ref):
    cnt = cnt_ref[0, :, 0:1] + cnt_ref[1, :, 0:1]
    return 1.0 / jnp.maximum(cnt, 1.0)


def _combine_body(do, s_ref, cnt_ref, r_ref, inp_ref, wl_ref, bl_ref, wr_ref,
                  m_ref, rn_ref):
    agg = (s_ref[0] + s_ref[1]) * _inv_counts(cnt_ref)
    h = jnp.maximum(agg + r_ref[...], 0.0) + 0.2 * inp_ref[...]
    m_ref[...] = jnp.dot(h, wl_ref[...], preferred_element_type=jnp.float32)
    rn_ref[...] = jnp.dot(h, wr_ref[...],
                          preferred_element_type=jnp.float32) + bl_ref[...]


def _tc_combine(S, cntP, r, inp, Wl, bl, Wr):
    do = Wl.shape[1]
    row = pl.BlockSpec((RB, 128), lambda i: (i, 0))
    rowo = pl.BlockSpec((RB, do), lambda i: (i, 0))
    return pl.pallas_call(
        functools.partial(_combine_body, do),
        grid=(GRID,),
        in_specs=[
            pl.BlockSpec((2, RB, 128), lambda i: (0, i, 0)),
            pl.BlockSpec((2, RB, 128), lambda i: (0, i, 0)),
            row, row,
            pl.BlockSpec((128, do), lambda i: (0, 0)),
            pl.BlockSpec((1, do), lambda i: (0, 0)),
            pl.BlockSpec((128, do), lambda i: (0, 0)),
        ],
        out_specs=[rowo, rowo],
        out_shape=[jax.ShapeDtypeStruct((N, do), jnp.float32)] * 2,
    )(S, cntP, r, inp, Wl, bl.reshape(1, do), Wr)


def _final_body(s_ref, cnt_ref, r_ref, out_ref):
    zf = (s_ref[0] + s_ref[1]) * _inv_counts(cnt_ref) + r_ref[...]
    z = zf[:, 0:64]
    zmax = jnp.max(z, axis=-1, keepdims=True)
    ez = jnp.exp(z - zmax)
    lse = jnp.log(jnp.sum(ez, axis=-1, keepdims=True)) + zmax
    out_ref[...] = z - lse


def _tc_final(S, cntP, r):
    row = pl.BlockSpec((RB, 128), lambda i: (i, 0))
    return pl.pallas_call(
        _final_body,
        grid=(GRID,),
        in_specs=[
            pl.BlockSpec((2, RB, 128), lambda i: (0, i, 0)),
            pl.BlockSpec((2, RB, 128), lambda i: (0, i, 0)),
            row,
        ],
        out_specs=pl.BlockSpec((RB, 64), lambda i: (i, 0)),
        out_shape=jax.ShapeDtypeStruct((N, 64), jnp.float32),
    )(S, cntP, r)


def kernel(x, edge_index, W_in, b_in, Wl, bl, Wr):
    src = edge_index[0]
    dst = edge_index[1]
    dst3 = jnp.concatenate(
        [dst, jnp.full((EPADC - E,), NP - 1, jnp.int32)]
    ).reshape(NW, NSTEPC, EBC)
    zeros128 = jnp.zeros((RPW, 128), jnp.float32)
    ones128 = jnp.ones((EBC, 128), jnp.float32)
    Wls = [Wl[1], Wl[2], jnp.pad(Wl[3], ((0, 0), (0, 64)))]
    Wrs = [Wr[1], Wr[2], jnp.pad(Wr[3], ((0, 0), (0, 64)))]
    bls = [bl[1], bl[2], jnp.pad(bl[3], (0, 64))]

    cntP = _make_sc_count()(dst3, ones128, zeros128)
    inp, m, r = _tc_proj(x, W_in, b_in, Wl[0], bl[0], Wr[0])
    for i in range(3):
        S = _make_sc_agg(128)(m, src, dst, zeros128)
        m, r = _tc_combine(S, cntP, r, inp, Wls[i], bls[i], Wrs[i])
    S = _make_sc_agg(128)(m, src, dst, zeros128)
    return _tc_final(S, cntP, r)


# speedup vs baseline: 1.1307x; 1.0456x over previous
import functools

import jax
import jax.numpy as jnp
from jax import lax
from jax.experimental import pallas as pl
from jax.experimental.pallas import tpu as pltpu
from jax.experimental.pallas import tpu_sc as plsc

N = 10000
E = 320000
NC = 2
NS = 16
NW = NC * NS
EBA = 80
NSTEP0 = 188
NSTEP1 = 62
EBC = 128
NSTEPC = 80
EPADC = NW * NSTEPC * EBC
NP = 10240
RPW = NP // NS

@functools.lru_cache(maxsize=None)
def _mesh():
    return plsc.VectorSubcoreMesh(core_axis_name="c", subcore_axis_name="s",
                                  num_cores=NC, num_subcores=NS)


def _sc_agg_body(D, m_hbm, src_hbm, dst_hbm, zeros_hbm, out_hbm,
                 isA, isB, idA, idB, rowsA, rowsB, acc,
                 sgA, sgB, siA, siB):
    c = lax.axis_index("c")
    s = lax.axis_index("s")
    nst = jnp.where(c == 0, NSTEP0, NSTEP1)
    base = c * NS * NSTEP0 * EBA + s * nst * EBA

    pltpu.sync_copy(zeros_hbm, acc.at[pl.ds(s * RPW, RPW)])
    pltpu.sync_copy(src_hbm.at[pl.ds(base, EBA)], isA)
    pltpu.sync_copy(dst_hbm.at[pl.ds(base, EBA)], idA)
    pltpu.sync_copy(src_hbm.at[pl.ds(base + EBA, EBA)], isB)
    pltpu.sync_copy(dst_hbm.at[pl.ds(base + EBA, EBA)], idB)
    plsc.subcore_barrier()

    pltpu.async_copy(m_hbm.at[isA], rowsA, sgA)
    pltpu.async_copy(m_hbm.at[isB], rowsB, sgB)

    def pair(k, carry):
        b = 2 * k
        offA = base + (b + 2) * EBA
        offB = base + (b + 3) * EBA
        pltpu.make_async_copy(m_hbm.at[isA], rowsA, sgA).wait()
        pltpu.sync_copy(rowsA, acc.at[idA], add=True)
        pltpu.async_copy(src_hbm.at[pl.ds(offA, EBA)], isA, siA)
        pltpu.async_copy(dst_hbm.at[pl.ds(offA, EBA)], idA, siA)
        pltpu.make_async_copy(m_hbm.at[isB], rowsB, sgB).wait()
        pltpu.sync_copy(rowsB, acc.at[idB], add=True)
        pltpu.async_copy(src_hbm.at[pl.ds(offB, EBA)], isB, siB)
        pltpu.async_copy(dst_hbm.at[pl.ds(offB, EBA)], idB, siB)
        pltpu.make_async_copy(src_hbm.at[pl.ds(offA, EBA)], isA, siA).wait()
        pltpu.make_async_copy(dst_hbm.at[pl.ds(offA, EBA)], idA, siA).wait()
        pltpu.async_copy(m_hbm.at[isA], rowsA, sgA)
        pltpu.make_async_copy(src_hbm.at[pl.ds(offB, EBA)], isB, siB).wait()
        pltpu.make_async_copy(dst_hbm.at[pl.ds(offB, EBA)], idB, siB).wait()
        pltpu.async_copy(m_hbm.at[isB], rowsB, sgB)
        return carry

    lax.fori_loop(0, nst // 2 - 1, pair, 0)
    pltpu.make_async_copy(m_hbm.at[isA], rowsA, sgA).wait()
    pltpu.sync_copy(rowsA, acc.at[idA], add=True)
    pltpu.make_async_copy(m_hbm.at[isB], rowsB, sgB).wait()
    pltpu.sync_copy(rowsB, acc.at[idB], add=True)
    plsc.subcore_barrier()
    pltpu.sync_copy(acc.at[pl.ds(s * RPW, RPW)],
                    out_hbm.at[c, pl.ds(s * RPW, RPW)])


@functools.lru_cache(maxsize=None)
def _make_sc_agg(D):
    return pl.kernel(
        functools.partial(_sc_agg_body, D),
        out_type=jax.ShapeDtypeStruct((NC, NP, D), jnp.float32),
        mesh=_mesh(),
        scratch_types=[
            pltpu.VMEM((EBA,), jnp.int32),
            pltpu.VMEM((EBA,), jnp.int32),
            pltpu.VMEM((EBA,), jnp.int32),
            pltpu.VMEM((EBA,), jnp.int32),
            pltpu.VMEM((EBA, D), jnp.float32),
            pltpu.VMEM((EBA, D), jnp.float32),
            pltpu.VMEM_SHARED((NP, D), jnp.float32),
            pltpu.SemaphoreType.DMA,
            pltpu.SemaphoreType.DMA,
            pltpu.SemaphoreType.DMA,
            pltpu.SemaphoreType.DMA,
        ],
    )


def _sc_count_body(dst3_hbm, ones_hbm, zeros_hbm, out_hbm,
                   idx_d, ones_v, acc, ssA, ssB):
    c = lax.axis_index("c")
    s = lax.axis_index("s")
    wid = c * NS + s
    pltpu.sync_copy(zeros_hbm, acc.at[pl.ds(s * RPW, RPW)])
    pltpu.sync_copy(ones_hbm, ones_v)
    pltpu.sync_copy(dst3_hbm.at[wid], idx_d)
    plsc.subcore_barrier()

    pltpu.async_copy(ones_v, acc.at[idx_d.at[0]], ssA, add=True)
    pltpu.async_copy(ones_v, acc.at[idx_d.at[1]], ssB, add=True)

    def pair(k, carry):
        b = 2 * k
        pltpu.make_async_copy(ones_v, acc.at[idx_d.at[b]], ssA).wait()
        pltpu.async_copy(ones_v, acc.at[idx_d.at[b + 2]], ssA, add=True)
        pltpu.make_async_copy(ones_v, acc.at[idx_d.at[b]], ssB).wait()
        pltpu.async_copy(ones_v, acc.at[idx_d.at[b + 3]], ssB, add=True)
        return carry

    lax.fori_loop(0, NSTEPC // 2 - 1, pair, 0)
    pltpu.make_async_copy(ones_v, acc.at[idx_d.at[0]], ssA).wait()
    pltpu.make_async_copy(ones_v, acc.at[idx_d.at[0]], ssB).wait()
    plsc.subcore_barrier()
    pltpu.sync_copy(acc.at[pl.ds(s * RPW, RPW)],
                    out_hbm.at[c, pl.ds(s * RPW, RPW)])


@functools.lru_cache(maxsize=None)
def _make_sc_count():
    return pl.kernel(
        _sc_count_body,
        out_type=jax.ShapeDtypeStruct((NC, NP, 128), jnp.float32),
        mesh=_mesh(),
        scratch_types=[
            pltpu.VMEM((NSTEPC, EBC), jnp.int32),
            pltpu.VMEM((EBC, 128), jnp.float32),
            pltpu.VMEM_SHARED((NP, 128), jnp.float32),
            pltpu.SemaphoreType.DMA,
            pltpu.SemaphoreType.DMA,
        ],
    )


RB = 1000
GRID = N // RB


def _proj_body(x_ref, win_ref, bin_ref, wl_ref, bl_ref, wr_ref,
               inp_ref, m_ref, r_ref):
    inp = jnp.dot(x_ref[...], win_ref[...],
                  preferred_element_type=jnp.float32) + bin_ref[...]
    h = jnp.maximum(inp, 0.0)
    inp_ref[...] = inp
    m_ref[...] = jnp.dot(h, wl_ref[...], preferred_element_type=jnp.float32)
    r_ref[...] = jnp.dot(h, wr_ref[...],
                         preferred_element_type=jnp.float32) + bl_ref[...]


def _tc_proj(x, W_in, b_in, Wl0, bl0, Wr0):
    row = pl.BlockSpec((RB, 128), lambda i: (i, 0))
    full = pl.BlockSpec((128, 128), lambda i: (0, 0))
    bias = pl.BlockSpec((1, 128), lambda i: (0, 0))
    return pl.pallas_call(
        _proj_body,
        grid=(GRID,),
        in_specs=[row, full, bias, full, bias, full],
        out_specs=[row, row, row],
        out_shape=[jax.ShapeDtypeStruct((N, 128), jnp.float32)] * 3,
    )(x, W_in, b_in.reshape(1, 128), Wl0, bl0.reshape(1, 128), Wr0)


def _inv_counts(cnt_ref):
    cnt = cnt_ref[0, :, 0:1] + cnt_ref[1, :, 0:1]
    return 1.0 / jnp.maximum(cnt, 1.0)


def _combine_body(do, s_ref, cnt_ref, r_ref, inp_ref, wl_ref, bl_ref, wr_ref,
                  m_ref, rn_ref):
    agg = (s_ref[0] + s_ref[1]) * _inv_counts(cnt_ref)
    h = jnp.maximum(agg + r_ref[...], 0.0) + 0.2 * inp_ref[...]
    m_ref[...] = jnp.dot(h, wl_ref[...], preferred_element_type=jnp.float32)
    rn_ref[...] = jnp.dot(h, wr_ref[...],
                          preferred_element_type=jnp.float32) + bl_ref[...]


def _tc_combine(S, cntP, r, inp, Wl, bl, Wr):
    do = Wl.shape[1]
    row = pl.BlockSpec((RB, 128), lambda i: (i, 0))
    rowo = pl.BlockSpec((RB, do), lambda i: (i, 0))
    return pl.pallas_call(
        functools.partial(_combine_body, do),
        grid=(GRID,),
        in_specs=[
            pl.BlockSpec((2, RB, 128), lambda i: (0, i, 0)),
            pl.BlockSpec((2, RB, 128), lambda i: (0, i, 0)),
            row, row,
            pl.BlockSpec((128, do), lambda i: (0, 0)),
            pl.BlockSpec((1, do), lambda i: (0, 0)),
            pl.BlockSpec((128, do), lambda i: (0, 0)),
        ],
        out_specs=[rowo, rowo],
        out_shape=[jax.ShapeDtypeStruct((N, do), jnp.float32)] * 2,
    )(S, cntP, r, inp, Wl, bl.reshape(1, do), Wr)


def _final_body(s_ref, cnt_ref, r_ref, out_ref):
    zf = (s_ref[0] + s_ref[1]) * _inv_counts(cnt_ref) + r_ref[...]
    z = zf[:, 0:64]
    zmax = jnp.max(z, axis=-1, keepdims=True)
    ez = jnp.exp(z - zmax)
    lse = jnp.log(jnp.sum(ez, axis=-1, keepdims=True)) + zmax
    out_ref[...] = z - lse


def _tc_final(S, cntP, r):
    row = pl.BlockSpec((RB, 128), lambda i: (i, 0))
    return pl.pallas_call(
        _final_body,
        grid=(GRID,),
        in_specs=[
            pl.BlockSpec((2, RB, 128), lambda i: (0, i, 0)),
            pl.BlockSpec((2, RB, 128), lambda i: (0, i, 0)),
            row,
        ],
        out_specs=pl.BlockSpec((RB, 64), lambda i: (i, 0)),
        out_shape=jax.ShapeDtypeStruct((N, 64), jnp.float32),
    )(S, cntP, r)


def kernel(x, edge_index, W_in, b_in, Wl, bl, Wr):
    src = edge_index[0]
    dst = edge_index[1]
    dst3 = jnp.concatenate(
        [dst, jnp.full((EPADC - E,), NP - 1, jnp.int32)]
    ).reshape(NW, NSTEPC, EBC)
    zeros128 = jnp.zeros((RPW, 128), jnp.float32)
    ones128 = jnp.ones((EBC, 128), jnp.float32)
    Wls = [Wl[1], Wl[2], jnp.pad(Wl[3], ((0, 0), (0, 64)))]
    Wrs = [Wr[1], Wr[2], jnp.pad(Wr[3], ((0, 0), (0, 64)))]
    bls = [bl[1], bl[2], jnp.pad(bl[3], (0, 64))]

    cntP = _make_sc_count()(dst3, ones128, zeros128)
    inp, m, r = _tc_proj(x, W_in, b_in, Wl[0], bl[0], Wr[0])
    for i in range(3):
        S = _make_sc_agg(128)(m, src, dst, zeros128)
        m, r = _tc_combine(S, cntP, r, inp, Wls[i], bls[i], Wrs[i])
    S = _make_sc_agg(128)(m, src, dst, zeros128)
    return _tc_final(S, cntP, r)


# speedup vs baseline: 1.1885x; 1.0511x over previous
import functools

import jax
import jax.numpy as jnp
from jax import lax
from jax.experimental import pallas as pl
from jax.experimental.pallas import tpu as pltpu
from jax.experimental.pallas import tpu_sc as plsc

N = 10000
E = 320000
NC = 2
NS = 16
NW = NC * NS
EBA = 80
NSTEP0 = 176
NSTEP1 = 74
EBC = 128
NSTEPC = 80
EPADC = NW * NSTEPC * EBC
NP = 10240
RPW = NP // NS

@functools.lru_cache(maxsize=None)
def _mesh():
    return plsc.VectorSubcoreMesh(core_axis_name="c", subcore_axis_name="s",
                                  num_cores=NC, num_subcores=NS)


def _sc_agg_body(D, m_hbm, src_hbm, dst_hbm, zeros_hbm, out_hbm,
                 isA, isB, idA, idB, rowsA, rowsB, acc,
                 sgA, sgB, siA, siB):
    c = lax.axis_index("c")
    s = lax.axis_index("s")
    nst = jnp.where(c == 0, NSTEP0, NSTEP1)
    base = c * NS * NSTEP0 * EBA + s * nst * EBA

    pltpu.sync_copy(zeros_hbm, acc.at[pl.ds(s * RPW, RPW)])
    pltpu.sync_copy(src_hbm.at[pl.ds(base, EBA)], isA)
    pltpu.sync_copy(dst_hbm.at[pl.ds(base, EBA)], idA)
    pltpu.sync_copy(src_hbm.at[pl.ds(base + EBA, EBA)], isB)
    pltpu.sync_copy(dst_hbm.at[pl.ds(base + EBA, EBA)], idB)
    plsc.subcore_barrier()

    pltpu.async_copy(m_hbm.at[isA], rowsA, sgA)
    pltpu.async_copy(m_hbm.at[isB], rowsB, sgB)

    def pair(k, carry):
        b = 2 * k
        offA = base + (b + 2) * EBA
        offB = base + (b + 3) * EBA
        pltpu.make_async_copy(m_hbm.at[isA], rowsA, sgA).wait()
        pltpu.sync_copy(rowsA, acc.at[idA], add=True)
        pltpu.async_copy(src_hbm.at[pl.ds(offA, EBA)], isA, siA)
        pltpu.async_copy(dst_hbm.at[pl.ds(offA, EBA)], idA, siA)
        pltpu.make_async_copy(m_hbm.at[isB], rowsB, sgB).wait()
        pltpu.sync_copy(rowsB, acc.at[idB], add=True)
        pltpu.async_copy(src_hbm.at[pl.ds(offB, EBA)], isB, siB)
        pltpu.async_copy(dst_hbm.at[pl.ds(offB, EBA)], idB, siB)
        pltpu.make_async_copy(src_hbm.at[pl.ds(offA, EBA)], isA, siA).wait()
        pltpu.make_async_copy(dst_hbm.at[pl.ds(offA, EBA)], idA, siA).wait()
        pltpu.async_copy(m_hbm.at[isA], rowsA, sgA)
        pltpu.make_async_copy(src_hbm.at[pl.ds(offB, EBA)], isB, siB).wait()
        pltpu.make_async_copy(dst_hbm.at[pl.ds(offB, EBA)], idB, siB).wait()
        pltpu.async_copy(m_hbm.at[isB], rowsB, sgB)
        return carry

    lax.fori_loop(0, nst // 2 - 1, pair, 0)
    pltpu.make_async_copy(m_hbm.at[isA], rowsA, sgA).wait()
    pltpu.sync_copy(rowsA, acc.at[idA], add=True)
    pltpu.make_async_copy(m_hbm.at[isB], rowsB, sgB).wait()
    pltpu.sync_copy(rowsB, acc.at[idB], add=True)
    plsc.subcore_barrier()
    pltpu.sync_copy(acc.at[pl.ds(s * RPW, RPW)],
                    out_hbm.at[c, pl.ds(s * RPW, RPW)])


@functools.lru_cache(maxsize=None)
def _make_sc_agg(D):
    return pl.kernel(
        functools.partial(_sc_agg_body, D),
        out_type=jax.ShapeDtypeStruct((NC, NP, D), jnp.float32),
        mesh=_mesh(),
        scratch_types=[
            pltpu.VMEM((EBA,), jnp.int32),
            pltpu.VMEM((EBA,), jnp.int32),
            pltpu.VMEM((EBA,), jnp.int32),
            pltpu.VMEM((EBA,), jnp.int32),
            pltpu.VMEM((EBA, D), jnp.float32),
            pltpu.VMEM((EBA, D), jnp.float32),
            pltpu.VMEM_SHARED((NP, D), jnp.float32),
            pltpu.SemaphoreType.DMA,
            pltpu.SemaphoreType.DMA,
            pltpu.SemaphoreType.DMA,
            pltpu.SemaphoreType.DMA,
        ],
    )


def _sc_count_body(dst3_hbm, ones_hbm, zeros_hbm, out_hbm,
                   idx_d, ones_v, acc, ssA, ssB):
    c = lax.axis_index("c")
    s = lax.axis_index("s")
    wid = c * NS + s
    pltpu.sync_copy(zeros_hbm, acc.at[pl.ds(s * RPW, RPW)])
    pltpu.sync_copy(ones_hbm, ones_v)
    pltpu.sync_copy(dst3_hbm.at[wid], idx_d)
    plsc.subcore_barrier()

    pltpu.async_copy(ones_v, acc.at[idx_d.at[0]], ssA, add=True)
    pltpu.async_copy(ones_v, acc.at[idx_d.at[1]], ssB, add=True)

    def pair(k, carry):
        b = 2 * k
        pltpu.make_async_copy(ones_v, acc.at[idx_d.at[b]], ssA).wait()
        pltpu.async_copy(ones_v, acc.at[idx_d.at[b + 2]], ssA, add=True)
        pltpu.make_async_copy(ones_v, acc.at[idx_d.at[b]], ssB).wait()
        pltpu.async_copy(ones_v, acc.at[idx_d.at[b + 3]], ssB, add=True)
        return carry

    lax.fori_loop(0, NSTEPC // 2 - 1, pair, 0)
    pltpu.make_async_copy(ones_v, acc.at[idx_d.at[0]], ssA).wait()
    pltpu.make_async_copy(ones_v, acc.at[idx_d.at[0]], ssB).wait()
    plsc.subcore_barrier()
    pltpu.sync_copy(acc.at[pl.ds(s * RPW, RPW)],
                    out_hbm.at[c, pl.ds(s * RPW, RPW)])


@functools.lru_cache(maxsize=None)
def _make_sc_count():
    return pl.kernel(
        _sc_count_body,
        out_type=jax.ShapeDtypeStruct((NC, NP, 128), jnp.float32),
        mesh=_mesh(),
        scratch_types=[
            pltpu.VMEM((NSTEPC, EBC), jnp.int32),
            pltpu.VMEM((EBC, 128), jnp.float32),
            pltpu.VMEM_SHARED((NP, 128), jnp.float32),
            pltpu.SemaphoreType.DMA,
            pltpu.SemaphoreType.DMA,
        ],
    )


RB = 1000
GRID = N // RB


def _proj_body(x_ref, win_ref, bin_ref, wl_ref, bl_ref, wr_ref,
               inp_ref, m_ref, r_ref):
    inp = jnp.dot(x_ref[...], win_ref[...],
                  preferred_element_type=jnp.float32) + bin_ref[...]
    h = jnp.maximum(inp, 0.0)
    inp_ref[...] = inp
    m_ref[...] = jnp.dot(h, wl_ref[...], preferred_element_type=jnp.float32)
    r_ref[...] = jnp.dot(h, wr_ref[...],
                         preferred_element_type=jnp.float32) + bl_ref[...]


def _tc_proj(x, W_in, b_in, Wl0, bl0, Wr0):
    row = pl.BlockSpec((RB, 128), lambda i: (i, 0))
    full = pl.BlockSpec((128, 128), lambda i: (0, 0))
    bias = pl.BlockSpec((1, 128), lambda i: (0, 0))
    return pl.pallas_call(
        _proj_body,
        grid=(GRID,),
        in_specs=[row, full, bias, full, bias, full],
        out_specs=[row, row, row],
        out_shape=[jax.ShapeDtypeStruct((N, 128), jnp.float32)] * 3,
    )(x, W_in, b_in.reshape(1, 128), Wl0, bl0.reshape(1, 128), Wr0)


def _inv_counts(cnt_ref):
    cnt = cnt_ref[0, :, 0:1] + cnt_ref[1, :, 0:1]
    return 1.0 / jnp.maximum(cnt, 1.0)


def _combine_body(do, s_ref, cnt_ref, r_ref, inp_ref, wl_ref, bl_ref, wr_ref,
                  m_ref, rn_ref):
    agg = (s_ref[0] + s_ref[1]) * _inv_counts(cnt_ref)
    h = jnp.maximum(agg + r_ref[...], 0.0) + 0.2 * inp_ref[...]
    m_ref[...] = jnp.dot(h, wl_ref[...], preferred_element_type=jnp.float32)
    rn_ref[...] = jnp.dot(h, wr_ref[...],
                          preferred_element_type=jnp.float32) + bl_ref[...]


def _tc_combine(S, cntP, r, inp, Wl, bl, Wr):
    do = Wl.shape[1]
    row = pl.BlockSpec((RB, 128), lambda i: (i, 0))
    rowo = pl.BlockSpec((RB, do), lambda i: (i, 0))
    return pl.pallas_call(
        functools.partial(_combine_body, do),
        grid=(GRID,),
        in_specs=[
            pl.BlockSpec((2, RB, 128), lambda i: (0, i, 0)),
            pl.BlockSpec((2, RB, 128), lambda i: (0, i, 0)),
            row, row,
            pl.BlockSpec((128, do), lambda i: (0, 0)),
            pl.BlockSpec((1, do), lambda i: (0, 0)),
            pl.BlockSpec((128, do), lambda i: (0, 0)),
        ],
        out_specs=[rowo, rowo],
        out_shape=[jax.ShapeDtypeStruct((N, do), jnp.float32)] * 2,
    )(S, cntP, r, inp, Wl, bl.reshape(1, do), Wr)


def _final_body(s_ref, cnt_ref, r_ref, out_ref):
    zf = (s_ref[0] + s_ref[1]) * _inv_counts(cnt_ref) + r_ref[...]
    z = zf[:, 0:64]
    zmax = jnp.max(z, axis=-1, keepdims=True)
    ez = jnp.exp(z - zmax)
    lse = jnp.log(jnp.sum(ez, axis=-1, keepdims=True)) + zmax
    out_ref[...] = z - lse


def _tc_final(S, cntP, r):
    row = pl.BlockSpec((RB, 128), lambda i: (i, 0))
    return pl.pallas_call(
        _final_body,
        grid=(GRID,),
        in_specs=[
            pl.BlockSpec((2, RB, 128), lambda i: (0, i, 0)),
            pl.BlockSpec((2, RB, 128), lambda i: (0, i, 0)),
            row,
        ],
        out_specs=pl.BlockSpec((RB, 64), lambda i: (i, 0)),
        out_shape=jax.ShapeDtypeStruct((N, 64), jnp.float32),
    )(S, cntP, r)


def kernel(x, edge_index, W_in, b_in, Wl, bl, Wr):
    src = edge_index[0]
    dst = edge_index[1]
    dst3 = jnp.concatenate(
        [dst, jnp.full((EPADC - E,), NP - 1, jnp.int32)]
    ).reshape(NW, NSTEPC, EBC)
    zeros128 = jnp.zeros((RPW, 128), jnp.float32)
    ones128 = jnp.ones((EBC, 128), jnp.float32)
    Wls = [Wl[1], Wl[2], jnp.pad(Wl[3], ((0, 0), (0, 64)))]
    Wrs = [Wr[1], Wr[2], jnp.pad(Wr[3], ((0, 0), (0, 64)))]
    bls = [bl[1], bl[2], jnp.pad(bl[3], (0, 64))]

    cntP = _make_sc_count()(dst3, ones128, zeros128)
    inp, m, r = _tc_proj(x, W_in, b_in, Wl[0], bl[0], Wr[0])
    for i in range(3):
        S = _make_sc_agg(128)(m, src, dst, zeros128)
        m, r = _tc_combine(S, cntP, r, inp, Wls[i], bls[i], Wrs[i])
    S = _make_sc_agg(128)(m, src, dst, zeros128)
    return _tc_final(S, cntP, r)


# speedup vs baseline: 1.2487x; 1.0506x over previous
import functools

import jax
import jax.numpy as jnp
from jax import lax
from jax.experimental import pallas as pl
from jax.experimental.pallas import tpu as pltpu
from jax.experimental.pallas import tpu_sc as plsc

N = 10000
E = 320000
NC = 2
NS = 16
NW = NC * NS
EBA = 80
NSTEP0 = 164
NSTEP1 = 86
EBC = 128
NSTEPC = 80
EPADC = NW * NSTEPC * EBC
NP = 10240
RPW = NP // NS

@functools.lru_cache(maxsize=None)
def _mesh():
    return plsc.VectorSubcoreMesh(core_axis_name="c", subcore_axis_name="s",
                                  num_cores=NC, num_subcores=NS)


def _sc_agg_body(D, m_hbm, src_hbm, dst_hbm, zeros_hbm, out_hbm,
                 isA, isB, idA, idB, rowsA, rowsB, acc,
                 sgA, sgB, siA, siB):
    c = lax.axis_index("c")
    s = lax.axis_index("s")
    nst = jnp.where(c == 0, NSTEP0, NSTEP1)
    base = c * NS * NSTEP0 * EBA + s * nst * EBA

    pltpu.sync_copy(zeros_hbm, acc.at[pl.ds(s * RPW, RPW)])
    pltpu.sync_copy(src_hbm.at[pl.ds(base, EBA)], isA)
    pltpu.sync_copy(dst_hbm.at[pl.ds(base, EBA)], idA)
    pltpu.sync_copy(src_hbm.at[pl.ds(base + EBA, EBA)], isB)
    pltpu.sync_copy(dst_hbm.at[pl.ds(base + EBA, EBA)], idB)
    plsc.subcore_barrier()

    pltpu.async_copy(m_hbm.at[isA], rowsA, sgA)
    pltpu.async_copy(m_hbm.at[isB], rowsB, sgB)

    def pair(k, carry):
        b = 2 * k
        offA = base + (b + 2) * EBA
        offB = base + (b + 3) * EBA
        pltpu.make_async_copy(m_hbm.at[isA], rowsA, sgA).wait()
        pltpu.sync_copy(rowsA, acc.at[idA], add=True)
        pltpu.async_copy(src_hbm.at[pl.ds(offA, EBA)], isA, siA)
        pltpu.async_copy(dst_hbm.at[pl.ds(offA, EBA)], idA, siA)
        pltpu.make_async_copy(m_hbm.at[isB], rowsB, sgB).wait()
        pltpu.sync_copy(rowsB, acc.at[idB], add=True)
        pltpu.async_copy(src_hbm.at[pl.ds(offB, EBA)], isB, siB)
        pltpu.async_copy(dst_hbm.at[pl.ds(offB, EBA)], idB, siB)
        pltpu.make_async_copy(src_hbm.at[pl.ds(offA, EBA)], isA, siA).wait()
        pltpu.make_async_copy(dst_hbm.at[pl.ds(offA, EBA)], idA, siA).wait()
        pltpu.async_copy(m_hbm.at[isA], rowsA, sgA)
        pltpu.make_async_copy(src_hbm.at[pl.ds(offB, EBA)], isB, siB).wait()
        pltpu.make_async_copy(dst_hbm.at[pl.ds(offB, EBA)], idB, siB).wait()
        pltpu.async_copy(m_hbm.at[isB], rowsB, sgB)
        return carry

    lax.fori_loop(0, nst // 2 - 1, pair, 0)
    pltpu.make_async_copy(m_hbm.at[isA], rowsA, sgA).wait()
    pltpu.sync_copy(rowsA, acc.at[idA], add=True)
    pltpu.make_async_copy(m_hbm.at[isB], rowsB, sgB).wait()
    pltpu.sync_copy(rowsB, acc.at[idB], add=True)
    plsc.subcore_barrier()
    pltpu.sync_copy(acc.at[pl.ds(s * RPW, RPW)],
                    out_hbm.at[c, pl.ds(s * RPW, RPW)])


@functools.lru_cache(maxsize=None)
def _make_sc_agg(D):
    return pl.kernel(
        functools.partial(_sc_agg_body, D),
        out_type=jax.ShapeDtypeStruct((NC, NP, D), jnp.float32),
        mesh=_mesh(),
        scratch_types=[
            pltpu.VMEM((EBA,), jnp.int32),
            pltpu.VMEM((EBA,), jnp.int32),
            pltpu.VMEM((EBA,), jnp.int32),
            pltpu.VMEM((EBA,), jnp.int32),
            pltpu.VMEM((EBA, D), jnp.float32),
            pltpu.VMEM((EBA, D), jnp.float32),
            pltpu.VMEM_SHARED((NP, D), jnp.float32),
            pltpu.SemaphoreType.DMA,
            pltpu.SemaphoreType.DMA,
            pltpu.SemaphoreType.DMA,
            pltpu.SemaphoreType.DMA,
        ],
    )


def _sc_count_body(dst3_hbm, ones_hbm, zeros_hbm, out_hbm,
                   idx_d, ones_v, acc, ssA, ssB):
    c = lax.axis_index("c")
    s = lax.axis_index("s")
    wid = c * NS + s
    pltpu.sync_copy(zeros_hbm, acc.at[pl.ds(s * RPW, RPW)])
    pltpu.sync_copy(ones_hbm, ones_v)
    pltpu.sync_copy(dst3_hbm.at[wid], idx_d)
    plsc.subcore_barrier()

    pltpu.async_copy(ones_v, acc.at[idx_d.at[0]], ssA, add=True)
    pltpu.async_copy(ones_v, acc.at[idx_d.at[1]], ssB, add=True)

    def pair(k, carry):
        b = 2 * k
        pltpu.make_async_copy(ones_v, acc.at[idx_d.at[b]], ssA).wait()
        pltpu.async_copy(ones_v, acc.at[idx_d.at[b + 2]], ssA, add=True)
        pltpu.make_async_copy(ones_v, acc.at[idx_d.at[b]], ssB).wait()
        pltpu.async_copy(ones_v, acc.at[idx_d.at[b + 3]], ssB, add=True)
        return carry

    lax.fori_loop(0, NSTEPC // 2 - 1, pair, 0)
    pltpu.make_async_copy(ones_v, acc.at[idx_d.at[0]], ssA).wait()
    pltpu.make_async_copy(ones_v, acc.at[idx_d.at[0]], ssB).wait()
    plsc.subcore_barrier()
    pltpu.sync_copy(acc.at[pl.ds(s * RPW, RPW)],
                    out_hbm.at[c, pl.ds(s * RPW, RPW)])


@functools.lru_cache(maxsize=None)
def _make_sc_count():
    return pl.kernel(
        _sc_count_body,
        out_type=jax.ShapeDtypeStruct((NC, NP, 128), jnp.float32),
        mesh=_mesh(),
        scratch_types=[
            pltpu.VMEM((NSTEPC, EBC), jnp.int32),
            pltpu.VMEM((EBC, 128), jnp.float32),
            pltpu.VMEM_SHARED((NP, 128), jnp.float32),
            pltpu.SemaphoreType.DMA,
            pltpu.SemaphoreType.DMA,
        ],
    )


RB = 1000
GRID = N // RB


def _proj_body(x_ref, win_ref, bin_ref, wl_ref, bl_ref, wr_ref,
               inp_ref, m_ref, r_ref):
    inp = jnp.dot(x_ref[...], win_ref[...],
                  preferred_element_type=jnp.float32) + bin_ref[...]
    h = jnp.maximum(inp, 0.0)
    inp_ref[...] = inp
    m_ref[...] = jnp.dot(h, wl_ref[...], preferred_element_type=jnp.float32)
    r_ref[...] = jnp.dot(h, wr_ref[...],
                         preferred_element_type=jnp.float32) + bl_ref[...]


def _tc_proj(x, W_in, b_in, Wl0, bl0, Wr0):
    row = pl.BlockSpec((RB, 128), lambda i: (i, 0))
    full = pl.BlockSpec((128, 128), lambda i: (0, 0))
    bias = pl.BlockSpec((1, 128), lambda i: (0, 0))
    return pl.pallas_call(
        _proj_body,
        grid=(GRID,),
        in_specs=[row, full, bias, full, bias, full],
        out_specs=[row, row, row],
        out_shape=[jax.ShapeDtypeStruct((N, 128), jnp.float32)] * 3,
    )(x, W_in, b_in.reshape(1, 128), Wl0, bl0.reshape(1, 128), Wr0)


def _inv_counts(cnt_ref):
    cnt = cnt_ref[0, :, 0:1] + cnt_ref[1, :, 0:1]
    return 1.0 / jnp.maximum(cnt, 1.0)


def _combine_body(do, s_ref, cnt_ref, r_ref, inp_ref, wl_ref, bl_ref, wr_ref,
                  m_ref, rn_ref):
    agg = (s_ref[0] + s_ref[1]) * _inv_counts(cnt_ref)
    h = jnp.maximum(agg + r_ref[...], 0.0) + 0.2 * inp_ref[...]
    m_ref[...] = jnp.dot(h, wl_ref[...], preferred_element_type=jnp.float32)
    rn_ref[...] = jnp.dot(h, wr_ref[...],
                          preferred_element_type=jnp.float32) + bl_ref[...]


def _tc_combine(S, cntP, r, inp, Wl, bl, Wr):
    do = Wl.shape[1]
    row = pl.BlockSpec((RB, 128), lambda i: (i, 0))
    rowo = pl.BlockSpec((RB, do), lambda i: (i, 0))
    return pl.pallas_call(
        functools.partial(_combine_body, do),
        grid=(GRID,),
        in_specs=[
            pl.BlockSpec((2, RB, 128), lambda i: (0, i, 0)),
            pl.BlockSpec((2, RB, 128), lambda i: (0, i, 0)),
            row, row,
            pl.BlockSpec((128, do), lambda i: (0, 0)),
            pl.BlockSpec((1, do), lambda i: (0, 0)),
            pl.BlockSpec((128, do), lambda i: (0, 0)),
        ],
        out_specs=[rowo, rowo],
        out_shape=[jax.ShapeDtypeStruct((N, do), jnp.float32)] * 2,
    )(S, cntP, r, inp, Wl, bl.reshape(1, do), Wr)


def _final_body(s_ref, cnt_ref, r_ref, out_ref):
    zf = (s_ref[0] + s_ref[1]) * _inv_counts(cnt_ref) + r_ref[...]
    z = zf[:, 0:64]
    zmax = jnp.max(z, axis=-1, keepdims=True)
    ez = jnp.exp(z - zmax)
    lse = jnp.log(jnp.sum(ez, axis=-1, keepdims=True)) + zmax
    out_ref[...] = z - lse


def _tc_final(S, cntP, r):
    row = pl.BlockSpec((RB, 128), lambda i: (i, 0))
    return pl.pallas_call(
        _final_body,
        grid=(GRID,),
        in_specs=[
            pl.BlockSpec((2, RB, 128), lambda i: (0, i, 0)),
            pl.BlockSpec((2, RB, 128), lambda i: (0, i, 0)),
            row,
        ],
        out_specs=pl.BlockSpec((RB, 64), lambda i: (i, 0)),
        out_shape=jax.ShapeDtypeStruct((N, 64), jnp.float32),
    )(S, cntP, r)


def kernel(x, edge_index, W_in, b_in, Wl, bl, Wr):
    src = edge_index[0]
    dst = edge_index[1]
    dst3 = jnp.concatenate(
        [dst, jnp.full((EPADC - E,), NP - 1, jnp.int32)]
    ).reshape(NW, NSTEPC, EBC)
    zeros128 = jnp.zeros((RPW, 128), jnp.float32)
    ones128 = jnp.ones((EBC, 128), jnp.float32)
    Wls = [Wl[1], Wl[2], jnp.pad(Wl[3], ((0, 0), (0, 64)))]
    Wrs = [Wr[1], Wr[2], jnp.pad(Wr[3], ((0, 0), (0, 64)))]
    bls = [bl[1], bl[2], jnp.pad(bl[3], (0, 64))]

    cntP = _make_sc_count()(dst3, ones128, zeros128)
    inp, m, r = _tc_proj(x, W_in, b_in, Wl[0], bl[0], Wr[0])
    for i in range(3):
        S = _make_sc_agg(128)(m, src, dst, zeros128)
        m, r = _tc_combine(S, cntP, r, inp, Wls[i], bls[i], Wrs[i])
    S = _make_sc_agg(128)(m, src, dst, zeros128)
    return _tc_final(S, cntP, r)


# speedup vs baseline: 1.3283x; 1.0638x over previous
import functools

import jax
import jax.numpy as jnp
from jax import lax
from jax.experimental import pallas as pl
from jax.experimental.pallas import tpu as pltpu
from jax.experimental.pallas import tpu_sc as plsc

N = 10000
E = 320000
NC = 2
NS = 16
NW = NC * NS
EBA = 80
NSTEP0 = 150
NSTEP1 = 100
EBC = 128
NSTEPC = 80
EPADC = NW * NSTEPC * EBC
NP = 10240
RPW = NP // NS

@functools.lru_cache(maxsize=None)
def _mesh():
    return plsc.VectorSubcoreMesh(core_axis_name="c", subcore_axis_name="s",
                                  num_cores=NC, num_subcores=NS)


def _sc_agg_body(D, m_hbm, src_hbm, dst_hbm, zeros_hbm, out_hbm,
                 isA, isB, idA, idB, rowsA, rowsB, acc,
                 sgA, sgB, siA, siB):
    c = lax.axis_index("c")
    s = lax.axis_index("s")
    nst = jnp.where(c == 0, NSTEP0, NSTEP1)
    base = c * NS * NSTEP0 * EBA + s * nst * EBA

    pltpu.sync_copy(zeros_hbm, acc.at[pl.ds(s * RPW, RPW)])
    pltpu.sync_copy(src_hbm.at[pl.ds(base, EBA)], isA)
    pltpu.sync_copy(dst_hbm.at[pl.ds(base, EBA)], idA)
    pltpu.sync_copy(src_hbm.at[pl.ds(base + EBA, EBA)], isB)
    pltpu.sync_copy(dst_hbm.at[pl.ds(base + EBA, EBA)], idB)
    plsc.subcore_barrier()

    pltpu.async_copy(m_hbm.at[isA], rowsA, sgA)
    pltpu.async_copy(m_hbm.at[isB], rowsB, sgB)

    def pair(k, carry):
        b = 2 * k
        offA = base + (b + 2) * EBA
        offB = base + (b + 3) * EBA
        pltpu.make_async_copy(m_hbm.at[isA], rowsA, sgA).wait()
        pltpu.sync_copy(rowsA, acc.at[idA], add=True)
        pltpu.async_copy(src_hbm.at[pl.ds(offA, EBA)], isA, siA)
        pltpu.async_copy(dst_hbm.at[pl.ds(offA, EBA)], idA, siA)
        pltpu.make_async_copy(m_hbm.at[isB], rowsB, sgB).wait()
        pltpu.sync_copy(rowsB, acc.at[idB], add=True)
        pltpu.async_copy(src_hbm.at[pl.ds(offB, EBA)], isB, siB)
        pltpu.async_copy(dst_hbm.at[pl.ds(offB, EBA)], idB, siB)
        pltpu.make_async_copy(src_hbm.at[pl.ds(offA, EBA)], isA, siA).wait()
        pltpu.make_async_copy(dst_hbm.at[pl.ds(offA, EBA)], idA, siA).wait()
        pltpu.async_copy(m_hbm.at[isA], rowsA, sgA)
        pltpu.make_async_copy(src_hbm.at[pl.ds(offB, EBA)], isB, siB).wait()
        pltpu.make_async_copy(dst_hbm.at[pl.ds(offB, EBA)], idB, siB).wait()
        pltpu.async_copy(m_hbm.at[isB], rowsB, sgB)
        return carry

    lax.fori_loop(0, nst // 2 - 1, pair, 0)
    pltpu.make_async_copy(m_hbm.at[isA], rowsA, sgA).wait()
    pltpu.sync_copy(rowsA, acc.at[idA], add=True)
    pltpu.make_async_copy(m_hbm.at[isB], rowsB, sgB).wait()
    pltpu.sync_copy(rowsB, acc.at[idB], add=True)
    plsc.subcore_barrier()
    pltpu.sync_copy(acc.at[pl.ds(s * RPW, RPW)],
                    out_hbm.at[c, pl.ds(s * RPW, RPW)])


@functools.lru_cache(maxsize=None)
def _make_sc_agg(D):
    return pl.kernel(
        functools.partial(_sc_agg_body, D),
        out_type=jax.ShapeDtypeStruct((NC, NP, D), jnp.float32),
        mesh=_mesh(),
        scratch_types=[
            pltpu.VMEM((EBA,), jnp.int32),
            pltpu.VMEM((EBA,), jnp.int32),
            pltpu.VMEM((EBA,), jnp.int32),
            pltpu.VMEM((EBA,), jnp.int32),
            pltpu.VMEM((EBA, D), jnp.float32),
            pltpu.VMEM((EBA, D), jnp.float32),
            pltpu.VMEM_SHARED((NP, D), jnp.float32),
            pltpu.SemaphoreType.DMA,
            pltpu.SemaphoreType.DMA,
            pltpu.SemaphoreType.DMA,
            pltpu.SemaphoreType.DMA,
        ],
    )


def _sc_count_body(dst3_hbm, ones_hbm, zeros_hbm, out_hbm,
                   idx_d, ones_v, acc, ssA, ssB):
    c = lax.axis_index("c")
    s = lax.axis_index("s")
    wid = c * NS + s
    pltpu.sync_copy(zeros_hbm, acc.at[pl.ds(s * RPW, RPW)])
    pltpu.sync_copy(ones_hbm, ones_v)
    pltpu.sync_copy(dst3_hbm.at[wid], idx_d)
    plsc.subcore_barrier()

    pltpu.async_copy(ones_v, acc.at[idx_d.at[0]], ssA, add=True)
    pltpu.async_copy(ones_v, acc.at[idx_d.at[1]], ssB, add=True)

    def pair(k, carry):
        b = 2 * k
        pltpu.make_async_copy(ones_v, acc.at[idx_d.at[b]], ssA).wait()
        pltpu.async_copy(ones_v, acc.at[idx_d.at[b + 2]], ssA, add=True)
        pltpu.make_async_copy(ones_v, acc.at[idx_d.at[b]], ssB).wait()
        pltpu.async_copy(ones_v, acc.at[idx_d.at[b + 3]], ssB, add=True)
        return carry

    lax.fori_loop(0, NSTEPC // 2 - 1, pair, 0)
    pltpu.make_async_copy(ones_v, acc.at[idx_d.at[0]], ssA).wait()
    pltpu.make_async_copy(ones_v, acc.at[idx_d.at[0]], ssB).wait()
    plsc.subcore_barrier()
    pltpu.sync_copy(acc.at[pl.ds(s * RPW, RPW)],
                    out_hbm.at[c, pl.ds(s * RPW, RPW)])


@functools.lru_cache(maxsize=None)
def _make_sc_count():
    return pl.kernel(
        _sc_count_body,
        out_type=jax.ShapeDtypeStruct((NC, NP, 128), jnp.float32),
        mesh=_mesh(),
        scratch_types=[
            pltpu.VMEM((NSTEPC, EBC), jnp.int32),
            pltpu.VMEM((EBC, 128), jnp.float32),
            pltpu.VMEM_SHARED((NP, 128), jnp.float32),
            pltpu.SemaphoreType.DMA,
            pltpu.SemaphoreType.DMA,
        ],
    )


RB = 1000
GRID = N // RB


def _proj_body(x_ref, win_ref, bin_ref, wl_ref, bl_ref, wr_ref,
               inp_ref, m_ref, r_ref):
    inp = jnp.dot(x_ref[...], win_ref[...],
                  preferred_element_type=jnp.float32) + bin_ref[...]
    h = jnp.maximum(inp, 0.0)
    inp_ref[...] = inp
    m_ref[...] = jnp.dot(h, wl_ref[...], preferred_element_type=jnp.float32)
    r_ref[...] = jnp.dot(h, wr_ref[...],
                         preferred_element_type=jnp.float32) + bl_ref[...]


def _tc_proj(x, W_in, b_in, Wl0, bl0, Wr0):
    row = pl.BlockSpec((RB, 128), lambda i: (i, 0))
    full = pl.BlockSpec((128, 128), lambda i: (0, 0))
    bias = pl.BlockSpec((1, 128), lambda i: (0, 0))
    return pl.pallas_call(
        _proj_body,
        grid=(GRID,),
        in_specs=[row, full, bias, full, bias, full],
        out_specs=[row, row, row],
        out_shape=[jax.ShapeDtypeStruct((N, 128), jnp.float32)] * 3,
    )(x, W_in, b_in.reshape(1, 128), Wl0, bl0.reshape(1, 128), Wr0)


def _inv_counts(cnt_ref):
    cnt = cnt_ref[0, :, 0:1] + cnt_ref[1, :, 0:1]
    return 1.0 / jnp.maximum(cnt, 1.0)


def _combine_body(do, s_ref, cnt_ref, r_ref, inp_ref, wl_ref, bl_ref, wr_ref,
                  m_ref, rn_ref):
    agg = (s_ref[0] + s_ref[1]) * _inv_counts(cnt_ref)
    h = jnp.maximum(agg + r_ref[...], 0.0) + 0.2 * inp_ref[...]
    m_ref[...] = jnp.dot(h, wl_ref[...], preferred_element_type=jnp.float32)
    rn_ref[...] = jnp.dot(h, wr_ref[...],
                          preferred_element_type=jnp.float32) + bl_ref[...]


def _tc_combine(S, cntP, r, inp, Wl, bl, Wr):
    do = Wl.shape[1]
    row = pl.BlockSpec((RB, 128), lambda i: (i, 0))
    rowo = pl.BlockSpec((RB, do), lambda i: (i, 0))
    return pl.pallas_call(
        functools.partial(_combine_body, do),
        grid=(GRID,),
        in_specs=[
            pl.BlockSpec((2, RB, 128), lambda i: (0, i, 0)),
            pl.BlockSpec((2, RB, 128), lambda i: (0, i, 0)),
            row, row,
            pl.BlockSpec((128, do), lambda i: (0, 0)),
            pl.BlockSpec((1, do), lambda i: (0, 0)),
            pl.BlockSpec((128, do), lambda i: (0, 0)),
        ],
        out_specs=[rowo, rowo],
        out_shape=[jax.ShapeDtypeStruct((N, do), jnp.float32)] * 2,
    )(S, cntP, r, inp, Wl, bl.reshape(1, do), Wr)


def _final_body(s_ref, cnt_ref, r_ref, out_ref):
    zf = (s_ref[0] + s_ref[1]) * _inv_counts(cnt_ref) + r_ref[...]
    z = zf[:, 0:64]
    zmax = jnp.max(z, axis=-1, keepdims=True)
    ez = jnp.exp(z - zmax)
    lse = jnp.log(jnp.sum(ez, axis=-1, keepdims=True)) + zmax
    out_ref[...] = z - lse


def _tc_final(S, cntP, r):
    row = pl.BlockSpec((RB, 128), lambda i: (i, 0))
    return pl.pallas_call(
        _final_body,
        grid=(GRID,),
        in_specs=[
            pl.BlockSpec((2, RB, 128), lambda i: (0, i, 0)),
            pl.BlockSpec((2, RB, 128), lambda i: (0, i, 0)),
            row,
        ],
        out_specs=pl.BlockSpec((RB, 64), lambda i: (i, 0)),
        out_shape=jax.ShapeDtypeStruct((N, 64), jnp.float32),
    )(S, cntP, r)


def kernel(x, edge_index, W_in, b_in, Wl, bl, Wr):
    src = edge_index[0]
    dst = edge_index[1]
    dst3 = jnp.concatenate(
        [dst, jnp.full((EPADC - E,), NP - 1, jnp.int32)]
    ).reshape(NW, NSTEPC, EBC)
    zeros128 = jnp.zeros((RPW, 128), jnp.float32)
    ones128 = jnp.ones((EBC, 128), jnp.float32)
    Wls = [Wl[1], Wl[2], jnp.pad(Wl[3], ((0, 0), (0, 64)))]
    Wrs = [Wr[1], Wr[2], jnp.pad(Wr[3], ((0, 0), (0, 64)))]
    bls = [bl[1], bl[2], jnp.pad(bl[3], (0, 64))]

    cntP = _make_sc_count()(dst3, ones128, zeros128)
    inp, m, r = _tc_proj(x, W_in, b_in, Wl[0], bl[0], Wr[0])
    for i in range(3):
        S = _make_sc_agg(128)(m, src, dst, zeros128)
        m, r = _tc_combine(S, cntP, r, inp, Wls[i], bls[i], Wrs[i])
    S = _make_sc_agg(128)(m, src, dst, zeros128)
    return _tc_final(S, cntP, r)


# speedup vs baseline: 1.4919x; 1.1231x over previous
import functools

import jax
import jax.numpy as jnp
from jax import lax
from jax.experimental import pallas as pl
from jax.experimental.pallas import tpu as pltpu
from jax.experimental.pallas import tpu_sc as plsc

N = 10000
E = 320000
NC = 2
NS = 16
NW = NC * NS
EBA = 80
NSTEP0 = 126
NSTEP1 = 124
EBC = 128
NSTEPC = 80
EPADC = NW * NSTEPC * EBC
NP = 10240
RPW = NP // NS

@functools.lru_cache(maxsize=None)
def _mesh():
    return plsc.VectorSubcoreMesh(core_axis_name="c", subcore_axis_name="s",
                                  num_cores=NC, num_subcores=NS)


def _sc_agg_body(D, m_hbm, src_hbm, dst_hbm, zeros_hbm, out_hbm,
                 isA, isB, idA, idB, rowsA, rowsB, acc,
                 sgA, sgB, siA, siB):
    c = lax.axis_index("c")
    s = lax.axis_index("s")
    nst = jnp.where(c == 0, NSTEP0, NSTEP1)
    base = c * NS * NSTEP0 * EBA + s * nst * EBA

    pltpu.sync_copy(zeros_hbm, acc.at[pl.ds(s * RPW, RPW)])
    pltpu.sync_copy(src_hbm.at[pl.ds(base, EBA)], isA)
    pltpu.sync_copy(dst_hbm.at[pl.ds(base, EBA)], idA)
    pltpu.sync_copy(src_hbm.at[pl.ds(base + EBA, EBA)], isB)
    pltpu.sync_copy(dst_hbm.at[pl.ds(base + EBA, EBA)], idB)
    plsc.subcore_barrier()

    pltpu.async_copy(m_hbm.at[isA], rowsA, sgA)
    pltpu.async_copy(m_hbm.at[isB], rowsB, sgB)

    def pair(k, carry):
        b = 2 * k
        offA = base + (b + 2) * EBA
        offB = base + (b + 3) * EBA
        pltpu.make_async_copy(m_hbm.at[isA], rowsA, sgA).wait()
        pltpu.sync_copy(rowsA, acc.at[idA], add=True)
        pltpu.async_copy(src_hbm.at[pl.ds(offA, EBA)], isA, siA)
        pltpu.async_copy(dst_hbm.at[pl.ds(offA, EBA)], idA, siA)
        pltpu.make_async_copy(m_hbm.at[isB], rowsB, sgB).wait()
        pltpu.sync_copy(rowsB, acc.at[idB], add=True)
        pltpu.async_copy(src_hbm.at[pl.ds(offB, EBA)], isB, siB)
        pltpu.async_copy(dst_hbm.at[pl.ds(offB, EBA)], idB, siB)
        pltpu.make_async_copy(src_hbm.at[pl.ds(offA, EBA)], isA, siA).wait()
        pltpu.make_async_copy(dst_hbm.at[pl.ds(offA, EBA)], idA, siA).wait()
        pltpu.async_copy(m_hbm.at[isA], rowsA, sgA)
        pltpu.make_async_copy(src_hbm.at[pl.ds(offB, EBA)], isB, siB).wait()
        pltpu.make_async_copy(dst_hbm.at[pl.ds(offB, EBA)], idB, siB).wait()
        pltpu.async_copy(m_hbm.at[isB], rowsB, sgB)
        return carry

    lax.fori_loop(0, nst // 2 - 1, pair, 0)
    pltpu.make_async_copy(m_hbm.at[isA], rowsA, sgA).wait()
    pltpu.sync_copy(rowsA, acc.at[idA], add=True)
    pltpu.make_async_copy(m_hbm.at[isB], rowsB, sgB).wait()
    pltpu.sync_copy(rowsB, acc.at[idB], add=True)
    plsc.subcore_barrier()
    pltpu.sync_copy(acc.at[pl.ds(s * RPW, RPW)],
                    out_hbm.at[c, pl.ds(s * RPW, RPW)])


@functools.lru_cache(maxsize=None)
def _make_sc_agg(D):
    return pl.kernel(
        functools.partial(_sc_agg_body, D),
        out_type=jax.ShapeDtypeStruct((NC, NP, D), jnp.float32),
        mesh=_mesh(),
        scratch_types=[
            pltpu.VMEM((EBA,), jnp.int32),
            pltpu.VMEM((EBA,), jnp.int32),
            pltpu.VMEM((EBA,), jnp.int32),
            pltpu.VMEM((EBA,), jnp.int32),
            pltpu.VMEM((EBA, D), jnp.float32),
            pltpu.VMEM((EBA, D), jnp.float32),
            pltpu.VMEM_SHARED((NP, D), jnp.float32),
            pltpu.SemaphoreType.DMA,
            pltpu.SemaphoreType.DMA,
            pltpu.SemaphoreType.DMA,
            pltpu.SemaphoreType.DMA,
        ],
    )


def _sc_count_body(dst3_hbm, ones_hbm, zeros_hbm, out_hbm,
                   idx_d, ones_v, acc, ssA, ssB):
    c = lax.axis_index("c")
    s = lax.axis_index("s")
    wid = c * NS + s
    pltpu.sync_copy(zeros_hbm, acc.at[pl.ds(s * RPW, RPW)])
    pltpu.sync_copy(ones_hbm, ones_v)
    pltpu.sync_copy(dst3_hbm.at[wid], idx_d)
    plsc.subcore_barrier()

    pltpu.async_copy(ones_v, acc.at[idx_d.at[0]], ssA, add=True)
    pltpu.async_copy(ones_v, acc.at[idx_d.at[1]], ssB, add=True)

    def pair(k, carry):
        b = 2 * k
        pltpu.make_async_copy(ones_v, acc.at[idx_d.at[b]], ssA).wait()
        pltpu.async_copy(ones_v, acc.at[idx_d.at[b + 2]], ssA, add=True)
        pltpu.make_async_copy(ones_v, acc.at[idx_d.at[b]], ssB).wait()
        pltpu.async_copy(ones_v, acc.at[idx_d.at[b + 3]], ssB, add=True)
        return carry

    lax.fori_loop(0, NSTEPC // 2 - 1, pair, 0)
    pltpu.make_async_copy(ones_v, acc.at[idx_d.at[0]], ssA).wait()
    pltpu.make_async_copy(ones_v, acc.at[idx_d.at[0]], ssB).wait()
    plsc.subcore_barrier()
    pltpu.sync_copy(acc.at[pl.ds(s * RPW, RPW)],
                    out_hbm.at[c, pl.ds(s * RPW, RPW)])


@functools.lru_cache(maxsize=None)
def _make_sc_count():
    return pl.kernel(
        _sc_count_body,
        out_type=jax.ShapeDtypeStruct((NC, NP, 128), jnp.float32),
        mesh=_mesh(),
        scratch_types=[
            pltpu.VMEM((NSTEPC, EBC), jnp.int32),
            pltpu.VMEM((EBC, 128), jnp.float32),
            pltpu.VMEM_SHARED((NP, 128), jnp.float32),
            pltpu.SemaphoreType.DMA,
            pltpu.SemaphoreType.DMA,
        ],
    )


RB = 1000
GRID = N // RB


def _proj_body(x_ref, win_ref, bin_ref, wl_ref, bl_ref, wr_ref,
               inp_ref, m_ref, r_ref):
    inp = jnp.dot(x_ref[...], win_ref[...],
                  preferred_element_type=jnp.float32) + bin_ref[...]
    h = jnp.maximum(inp, 0.0)
    inp_ref[...] = inp
    m_ref[...] = jnp.dot(h, wl_ref[...], preferred_element_type=jnp.float32)
    r_ref[...] = jnp.dot(h, wr_ref[...],
                         preferred_element_type=jnp.float32) + bl_ref[...]


def _tc_proj(x, W_in, b_in, Wl0, bl0, Wr0):
    row = pl.BlockSpec((RB, 128), lambda i: (i, 0))
    full = pl.BlockSpec((128, 128), lambda i: (0, 0))
    bias = pl.BlockSpec((1, 128), lambda i: (0, 0))
    return pl.pallas_call(
        _proj_body,
        grid=(GRID,),
        in_specs=[row, full, bias, full, bias, full],
        out_specs=[row, row, row],
        out_shape=[jax.ShapeDtypeStruct((N, 128), jnp.float32)] * 3,
    )(x, W_in, b_in.reshape(1, 128), Wl0, bl0.reshape(1, 128), Wr0)


def _inv_counts(cnt_ref):
    cnt = cnt_ref[0, :, 0:1] + cnt_ref[1, :, 0:1]
    return 1.0 / jnp.maximum(cnt, 1.0)


def _combine_body(do, s_ref, cnt_ref, r_ref, inp_ref, wl_ref, bl_ref, wr_ref,
                  m_ref, rn_ref):
    agg = (s_ref[0] + s_ref[1]) * _inv_counts(cnt_ref)
    h = jnp.maximum(agg + r_ref[...], 0.0) + 0.2 * inp_ref[...]
    m_ref[...] = jnp.dot(h, wl_ref[...], preferred_element_type=jnp.float32)
    rn_ref[...] = jnp.dot(h, wr_ref[...],
                          preferred_element_type=jnp.float32) + bl_ref[...]


def _tc_combine(S, cntP, r, inp, Wl, bl, Wr):
    do = Wl.shape[1]
    row = pl.BlockSpec((RB, 128), lambda i: (i, 0))
    rowo = pl.BlockSpec((RB, do), lambda i: (i, 0))
    return pl.pallas_call(
        functools.partial(_combine_body, do),
        grid=(GRID,),
        in_specs=[
            pl.BlockSpec((2, RB, 128), lambda i: (0, i, 0)),
            pl.BlockSpec((2, RB, 128), lambda i: (0, i, 0)),
            row, row,
            pl.BlockSpec((128, do), lambda i: (0, 0)),
            pl.BlockSpec((1, do), lambda i: (0, 0)),
            pl.BlockSpec((128, do), lambda i: (0, 0)),
        ],
        out_specs=[rowo, rowo],
        out_shape=[jax.ShapeDtypeStruct((N, do), jnp.float32)] * 2,
    )(S, cntP, r, inp, Wl, bl.reshape(1, do), Wr)


def _final_body(s_ref, cnt_ref, r_ref, out_ref):
    zf = (s_ref[0] + s_ref[1]) * _inv_counts(cnt_ref) + r_ref[...]
    z = zf[:, 0:64]
    zmax = jnp.max(z, axis=-1, keepdims=True)
    ez = jnp.exp(z - zmax)
    lse = jnp.log(jnp.sum(ez, axis=-1, keepdims=True)) + zmax
    out_ref[...] = z - lse


def _tc_final(S, cntP, r):
    row = pl.BlockSpec((RB, 128), lambda i: (i, 0))
    return pl.pallas_call(
        _final_body,
        grid=(GRID,),
        in_specs=[
            pl.BlockSpec((2, RB, 128), lambda i: (0, i, 0)),
            pl.BlockSpec((2, RB, 128), lambda i: (0, i, 0)),
            row,
        ],
        out_specs=pl.BlockSpec((RB, 64), lambda i: (i, 0)),
        out_shape=jax.ShapeDtypeStruct((N, 64), jnp.float32),
    )(S, cntP, r)


def kernel(x, edge_index, W_in, b_in, Wl, bl, Wr):
    src = edge_index[0]
    dst = edge_index[1]
    dst3 = jnp.concatenate(
        [dst, jnp.full((EPADC - E,), NP - 1, jnp.int32)]
    ).reshape(NW, NSTEPC, EBC)
    zeros128 = jnp.zeros((RPW, 128), jnp.float32)
    ones128 = jnp.ones((EBC, 128), jnp.float32)
    Wls = [Wl[1], Wl[2], jnp.pad(Wl[3], ((0, 0), (0, 64)))]
    Wrs = [Wr[1], Wr[2], jnp.pad(Wr[3], ((0, 0), (0, 64)))]
    bls = [bl[1], bl[2], jnp.pad(bl[3], (0, 64))]

    cntP = _make_sc_count()(dst3, ones128, zeros128)
    inp, m, r = _tc_proj(x, W_in, b_in, Wl[0], bl[0], Wr[0])
    for i in range(3):
        S = _make_sc_agg(128)(m, src, dst, zeros128)
        m, r = _tc_combine(S, cntP, r, inp, Wls[i], bls[i], Wrs[i])
    S = _make_sc_agg(128)(m, src, dst, zeros128)
    return _tc_final(S, cntP, r)
